# baseline (speedup 1.0000x reference)
import functools

import jax
import jax.numpy as jnp
from jax.experimental import pallas as pl
from jax.experimental.pallas import tpu as pltpu

_NBUF = 3


def _manual_kernel(x_hbm, w_ref, b_ref, o_ref, buf, sems, *,
                   cb, nchunks, half, inv_s):
    c = pl.program_id(0)
    base = c * half

    def start_copy(j, slot):
        pltpu.make_async_copy(
            x_hbm.at[pl.ds(base + j * cb, cb)],
            buf.at[slot],
            sems.at[slot],
        ).start()

    def wait_copy(slot):
        pltpu.make_async_copy(
            x_hbm.at[pl.ds(base, cb)],
            buf.at[slot],
            sems.at[slot],
        ).wait()

    for j in range(min(_NBUF, nchunks)):
        start_copy(j, j)

    for j in range(nchunks):
        slot = j % _NBUF
        wait_copy(slot)
        mean = jnp.sum(buf[slot], axis=1, dtype=jnp.float32) * inv_s
        y = jnp.dot(mean, w_ref[...], preferred_element_type=jnp.float32)
        o_ref[pl.ds(j * cb, cb), :] = (y + b_ref[...]).astype(o_ref.dtype)
        if j + _NBUF < nchunks:
            start_copy(j + _NBUF, slot)


def kernel(x, weight, bias):
    B, S, C_in = x.shape
    C_out = weight.shape[-1]
    out_dtype = x.dtype
    inv_s = 1.0 / float(S)

    half = B // 2
    cb = 32
    nchunks = half // cb

    buf_bytes = _NBUF * cb * S * C_in * 4
    vmem_limit = int(min(buf_bytes + (16 << 20), 110 << 20))

    cost = pl.CostEstimate(
        flops=B * S * C_in + 2 * B * C_in * C_out,
        transcendentals=0,
        bytes_accessed=x.size * 4 + weight.size * 4 + B * C_out * 4,
    )

    w = weight.astype(jnp.float32)
    b2d = bias.astype(jnp.float32).reshape(1, C_out)

    return pl.pallas_call(
        functools.partial(_manual_kernel, cb=cb, nchunks=nchunks,
                          half=half, inv_s=inv_s),
        out_shape=jax.ShapeDtypeStruct((B, C_out), out_dtype),
        grid=(2,),
        in_specs=[
            pl.BlockSpec(memory_space=pl.ANY),
            pl.BlockSpec((C_in, C_out), lambda c: (0, 0)),
            pl.BlockSpec((1, C_out), lambda c: (0, 0)),
        ],
        out_specs=pl.BlockSpec((half, C_out), lambda c: (c, 0)),
        scratch_shapes=[
            pltpu.VMEM((_NBUF, cb, S, C_in), jnp.float32),
            pltpu.SemaphoreType.DMA((_NBUF,)),
        ],
        compiler_params=pltpu.CompilerParams(
            dimension_semantics=("parallel",),
            vmem_limit_bytes=vmem_limit,
        ),
        cost_estimate=cost,
    )(x, w, b2d)


# speedup vs baseline: 1.0133x; 1.0133x over previous
import functools

import jax
import jax.numpy as jnp
from jax.experimental import pallas as pl
from jax.experimental.pallas import tpu as pltpu

_NBUF = 8


def _manual_kernel(x_hbm, w_ref, b_ref, o_ref, buf, sems, *,
                   cb, nchunks, half, inv_s):
    c = pl.program_id(0)
    base = c * half

    def start_copy(j, slot):
        pltpu.make_async_copy(
            x_hbm.at[pl.ds(base + j * cb, cb)],
            buf.at[slot],
            sems.at[slot],
        ).start()

    def wait_copy(slot):
        pltpu.make_async_copy(
            x_hbm.at[pl.ds(base, cb)],
            buf.at[slot],
            sems.at[slot],
        ).wait()

    for j in range(min(_NBUF, nchunks)):
        start_copy(j, j)

    for j in range(nchunks):
        slot = j % _NBUF
        wait_copy(slot)
        mean = jnp.sum(buf[slot], axis=1, dtype=jnp.float32) * inv_s
        y = jnp.dot(mean, w_ref[...], preferred_element_type=jnp.float32)
        o_ref[pl.ds(j * cb, cb), :] = (y + b_ref[...]).astype(o_ref.dtype)
        if j + _NBUF < nchunks:
            start_copy(j + _NBUF, slot)


def kernel(x, weight, bias):
    B, S, C_in = x.shape
    C_out = weight.shape[-1]
    out_dtype = x.dtype
    inv_s = 1.0 / float(S)

    half = B // 2
    cb = 16
    nchunks = half // cb

    buf_bytes = _NBUF * cb * S * C_in * 4
    vmem_limit = int(min(buf_bytes + (16 << 20), 110 << 20))

    cost = pl.CostEstimate(
        flops=B * S * C_in + 2 * B * C_in * C_out,
        transcendentals=0,
        bytes_accessed=x.size * 4 + weight.size * 4 + B * C_out * 4,
    )

    w = weight.astype(jnp.float32)
    b2d = bias.astype(jnp.float32).reshape(1, C_out)

    return pl.pallas_call(
        functools.partial(_manual_kernel, cb=cb, nchunks=nchunks,
                          half=half, inv_s=inv_s),
        out_shape=jax.ShapeDtypeStruct((B, C_out), out_dtype),
        grid=(2,),
        in_specs=[
            pl.BlockSpec(memory_space=pl.ANY),
            pl.BlockSpec((C_in, C_out), lambda c: (0, 0)),
            pl.BlockSpec((1, C_out), lambda c: (0, 0)),
        ],
        out_specs=pl.BlockSpec((half, C_out), lambda c: (c, 0)),
        scratch_shapes=[
            pltpu.VMEM((_NBUF, cb, S, C_in), jnp.float32),
            pltpu.SemaphoreType.DMA((_NBUF,)),
        ],
        compiler_params=pltpu.CompilerParams(
            dimension_semantics=("parallel",),
            vmem_limit_bytes=vmem_limit,
        ),
        cost_estimate=cost,
    )(x, w, b2d)


# speedup vs baseline: 1.0696x; 1.0556x over previous
import functools

import jax
import jax.numpy as jnp
from jax.experimental import pallas as pl
from jax.experimental.pallas import tpu as pltpu

_TB = 24


def _fused_kernel(x_ref, w_ref, b_ref, o_ref, *, inv_s):
    s = jnp.sum(x_ref[...], axis=1, dtype=jnp.float32)
    mean = s * inv_s
    y = jnp.dot(mean, w_ref[...], preferred_element_type=jnp.float32)
    o_ref[...] = (y + b_ref[...]).astype(o_ref.dtype)


def kernel(x, weight, bias):
    B, S, C_in = x.shape
    C_out = weight.shape[-1]
    out_dtype = x.dtype
    inv_s = 1.0 / float(S)
    itemsize = x.dtype.itemsize

    tb = _TB
    nb = -(-B // tb)

    x_block_bytes = tb * S * C_in * itemsize
    vmem_limit = int(min(2 * x_block_bytes + (8 << 20), 100 << 20))

    cost = pl.CostEstimate(
        flops=B * S * C_in + 2 * B * C_in * C_out,
        transcendentals=0,
        bytes_accessed=x.size * itemsize + weight.size * 4 + B * C_out * 4,
    )

    w = weight.astype(jnp.float32)
    b2d = bias.astype(jnp.float32).reshape(1, C_out)

    return pl.pallas_call(
        functools.partial(_fused_kernel, inv_s=inv_s),
        out_shape=jax.ShapeDtypeStruct((B, C_out), out_dtype),
        grid=(nb,),
        in_specs=[
            pl.BlockSpec((tb, S, C_in), lambda i: (i, 0, 0)),
            pl.BlockSpec((C_in, C_out), lambda i: (0, 0)),
            pl.BlockSpec((1, C_out), lambda i: (0, 0)),
        ],
        out_specs=pl.BlockSpec((tb, C_out), lambda i: (i, 0)),
        compiler_params=pltpu.CompilerParams(
            dimension_semantics=("parallel",),
            vmem_limit_bytes=vmem_limit,
        ),
        cost_estimate=cost,
    )(x, w, b2d)
